# Initial kernel scaffold; baseline (speedup 1.0000x reference)
import jax
import jax.numpy as jnp
from jax.experimental import pallas as pl


def kernel(user_node_id, recipe_node_id, recipe_x, edge_index_review, edge_label_index, user_emb, recipe_emb, lin_w, lin_b, c1u_Wl, c1u_bl, c1u_Wr, c1r_Wl, c1r_bl, c1r_Wr, c2u_Wl, c2u_bl, c2u_Wr, c2r_Wl, c2r_bl, c2r_Wr):
    raise NotImplementedError("write your pallas kernel here")


# speedup vs baseline: 2.6572x; 2.6572x over previous
import jax
import jax.numpy as jnp
from jax import lax
from jax.experimental import pallas as pl
from jax.experimental.pallas import tpu as pltpu
from jax.experimental.pallas import tpu_sc as plsc

H = 128
NU = 10000
NR = 10000
E = 320000
L = 100000

NC = 2
NS = 16

EK = 80
EPT = E // NS
ECH = EPT // EK
NPAD = 10240
RPT = NPAD // NS
assert EPT % EK == 0 and RPT % EK == 0

LP = 100352
LPT = LP // (NC * NS)
CK = 112
CCH = LPT // CK
assert LPT % CK == 0

_f32 = jnp.float32


def _zero_rows(ref, nrows, ncols):
    zero = jnp.zeros((16,), _f32)

    def body(r, c):
        for j in range(ncols // 16):
            ref[r, pl.ds(j * 16, 16)] = zero
        return c

    lax.fori_loop(0, nrows, body, 0)


def _fill_ones(ref, nrows):
    one = jnp.ones((16,), _f32)

    def body(r, c):
        ref[r, :] = one
        return c

    lax.fori_loop(0, nrows, body, 0)


def _make_seg(with_counts):
    def body(*refs):
        if with_counts:
            (x2, src_h, dst_h, agg2_h, cntp_h,
             rows_v, iv2, sidx_v, cnt_v, agg_s, sem) = refs
        else:
            (x2, src_h, dst_h, agg2_h,
             rows_v, iv2, sidx_v, agg_s, sem) = refs
            cntp_h = cnt_v = None

        cid = lax.axis_index("c")
        sid = lax.axis_index("s")

        _zero_rows(rows_v, EK, H)
        row0 = sid * RPT
        for j in range(RPT // EK):
            pltpu.sync_copy(rows_v, agg_s.at[pl.ds(row0 + j * EK, EK)])
        if with_counts:
            zero = jnp.zeros((16,), _f32)

            def zc(r, c):
                cnt_v[pl.ds(r * 16, 16)] = zero
                return c

            lax.fori_loop(0, NPAD // 16, zc, 0)
        plsc.subcore_barrier()

        base_e = sid * EPT

        def chunk(c, carry):
            off = base_e + c * EK
            pltpu.sync_copy(src_h.at[pl.ds(off, EK)], iv2.at[0])
            pltpu.sync_copy(dst_h.at[pl.ds(off, EK)], iv2.at[1])

            cidv = jnp.zeros((16,), jnp.int32) + cid
            one16 = jnp.ones((16,), _f32)

            def cpidx(k, carry2):
                sv16 = iv2[0, pl.ds(k * 16, 16)]
                dv16 = iv2[1, pl.ds(k * 16, 16)]
                sidx16 = sv16 + (dv16 - sv16) * cidv
                sidx_v[pl.ds(k * 16, 16)] = sidx16
                if with_counts:
                    plsc.addupdate_scatter(cnt_v, [sidx16], one16)
                return carry2

            lax.fori_loop(0, EK // 16, cpidx, 0)
            pltpu.async_copy(x2.at[1 - cid].at[iv2.at[1 - cid]], rows_v,
                             sem).wait()
            pltpu.sync_copy(rows_v, agg_s.at[sidx_v], add=True)
            return carry

        lax.fori_loop(0, ECH, chunk, 0)
        if with_counts:
            pltpu.sync_copy(cnt_v, cntp_h.at[cid, sid])
        plsc.subcore_barrier()

        for j in range(RPT // EK):
            r = row0 + j * EK
            pltpu.sync_copy(agg_s.at[pl.ds(r, EK)], rows_v)
            pltpu.sync_copy(rows_v, agg2_h.at[cid, pl.ds(r, EK)])

    out_type = [jax.ShapeDtypeStruct((2, NPAD, H), _f32)]
    scratch = [
        pltpu.VMEM((EK, H), _f32),
        pltpu.VMEM((2, EK), jnp.int32),
        pltpu.VMEM((EK,), jnp.int32),
    ]
    if with_counts:
        out_type += [jax.ShapeDtypeStruct((2, NS, NPAD), _f32)]
        scratch += [pltpu.VMEM((NPAD,), _f32)]
    scratch += [pltpu.VMEM_SHARED((NPAD, H), _f32)]
    scratch += [pltpu.SemaphoreType.DMA]
    return pl.kernel(
        body,
        out_type=out_type if with_counts else out_type[0],
        mesh=plsc.VectorSubcoreMesh(core_axis_name="c", subcore_axis_name="s"),
        scratch_types=scratch,
        compiler_params=pltpu.CompilerParams(needs_layout_passes=False),
        name="seg_sum_counts" if with_counts else "seg_sum",
    )


_seg_with_counts = _make_seg(True)
_seg_plain = _make_seg(False)


def _cls_body(h2_h, ia_h, ib_h, pred_h, a_v, b_v, iav, ibv, out_v,
              sem_a, sem_b):
    cid = lax.axis_index("c")
    sid = lax.axis_index("s")
    wid = sid * NC + cid
    base = wid * LPT

    def chunk(c, carry):
        off = base + c * CK
        pltpu.sync_copy(ia_h.at[pl.ds(off, CK)], iav)
        pltpu.sync_copy(ib_h.at[pl.ds(off, CK)], ibv)
        ca = pltpu.async_copy(h2_h.at[0].at[iav], a_v, sem_a)
        cb = pltpu.async_copy(h2_h.at[1].at[ibv], b_v, sem_b)
        ca.wait()
        cb.wait()

        def grp(g, carry2):
            rows = g * 16 + lax.iota(jnp.int32, 16)
            acc = jnp.zeros((16,), _f32)
            for f in range(H):
                cols = jnp.full((16,), f, jnp.int32)
                av = plsc.load_gather(a_v, [rows, cols])
                bv = plsc.load_gather(b_v, [rows, cols])
                acc = acc + av * bv
            out_v[pl.ds(c * CK + g * 16, 16)] = acc
            return carry2

        lax.fori_loop(0, CK // 16, grp, 0)
        return carry

    lax.fori_loop(0, CCH, chunk, 0)
    pltpu.sync_copy(out_v, pred_h.at[pl.ds(base, LPT)])


_cls = pl.kernel(
    _cls_body,
    out_type=jax.ShapeDtypeStruct((LP,), _f32),
    mesh=plsc.VectorSubcoreMesh(core_axis_name="c", subcore_axis_name="s"),
    scratch_types=[
        pltpu.VMEM((CK, H), _f32),
        pltpu.VMEM((CK, H), _f32),
        pltpu.VMEM((CK,), jnp.int32),
        pltpu.VMEM((CK,), jnp.int32),
        pltpu.VMEM((LPT,), _f32),
        pltpu.SemaphoreType.DMA,
        pltpu.SemaphoreType.DMA,
    ],
    compiler_params=pltpu.CompilerParams(needs_layout_passes=False),
    name="edge_dot_classifier",
)


BS = 1000


def _x2_body(ue_ref, rx_ref, remb_ref, lw_ref, lb_ref, o_ref):
    o_ref[0] = ue_ref[...]
    acc = lb_ref[...] + remb_ref[...]
    rx = rx_ref[...]
    lw = lw_ref[...]
    for k in range(10):
        acc = acc + rx[:, k:k + 1] * lw[k:k + 1, :]
    o_ref[1] = acc


_x2_proj = pl.pallas_call(
    _x2_body,
    grid=(NR // BS,),
    in_specs=[
        pl.BlockSpec((BS, H), lambda i: (i, 0)),
        pl.BlockSpec((BS, 10), lambda i: (i, 0)),
        pl.BlockSpec((BS, H), lambda i: (i, 0)),
        pl.BlockSpec((10, H), lambda i: (0, 0)),
        pl.BlockSpec((1, H), lambda i: (0, 0)),
    ],
    out_specs=pl.BlockSpec((2, BS, H), lambda i: (0, i, 0)),
    out_shape=jax.ShapeDtypeStruct((2, NR, H), _f32),
)


CB = 1024


def _cnt_inv_body(c_ref, o_ref):
    s = jnp.sum(c_ref[0], axis=0)
    o_ref[0] = (1.0 / jnp.maximum(s, 1.0))[:, None]


_cnt_inv = pl.pallas_call(
    _cnt_inv_body,
    grid=(2, NPAD // CB),
    in_specs=[pl.BlockSpec((1, NS, CB), lambda j, i: (j, 0, i))],
    out_specs=pl.BlockSpec((1, CB, 1), lambda j, i: (j, i, 0)),
    out_shape=jax.ShapeDtypeStruct((2, NPAD, 1), _f32),
)


def _make_layer(relu):
    def body(a_ref, x_ref, c_ref, wl_ref, bl_ref, wr_ref, o_ref):
        inv = c_ref[0]
        h = (jnp.dot(a_ref[0] * inv, wl_ref[0], preferred_element_type=_f32)
             + bl_ref[0]
             + jnp.dot(x_ref[0], wr_ref[0], preferred_element_type=_f32))
        if relu:
            h = jnp.maximum(h, 0.0)
        o_ref[0] = h

    return pl.pallas_call(
        body,
        grid=(2, NU // BS),
        in_specs=[
            pl.BlockSpec((1, BS, H), lambda j, i: (j, i, 0)),
            pl.BlockSpec((1, BS, H), lambda j, i: (j, i, 0)),
            pl.BlockSpec((1, BS, 1), lambda j, i: (j, i, 0)),
            pl.BlockSpec((1, H, H), lambda j, i: (j, 0, 0)),
            pl.BlockSpec((1, 1, H), lambda j, i: (j, 0, 0)),
            pl.BlockSpec((1, H, H), lambda j, i: (j, 0, 0)),
        ],
        out_specs=pl.BlockSpec((1, BS, H), lambda j, i: (j, i, 0)),
        out_shape=jax.ShapeDtypeStruct((2, NU, H), _f32),
    )


_layer_relu = _make_layer(True)
_layer_lin = _make_layer(False)


def kernel(user_node_id, recipe_node_id, recipe_x, edge_index_review,
           edge_label_index, user_emb, recipe_emb, lin_w, lin_b,
           c1u_Wl, c1u_bl, c1u_Wr, c1r_Wl, c1r_bl, c1r_Wr,
           c2u_Wl, c2u_bl, c2u_Wr, c2r_Wl, c2r_bl, c2r_Wr):
    src = edge_index_review[0]
    dst = edge_index_review[1]

    x2 = _x2_proj(user_emb, recipe_x, recipe_emb, lin_w, lin_b.reshape(1, H))

    wl1 = jnp.stack([c1u_Wl, c1r_Wl])
    bl1 = jnp.stack([c1u_bl, c1r_bl]).reshape(2, 1, H)
    wr1 = jnp.stack([c1u_Wr, c1r_Wr])
    wl2 = jnp.stack([c2u_Wl, c2r_Wl])
    bl2 = jnp.stack([c2u_bl, c2r_bl]).reshape(2, 1, H)
    wr2 = jnp.stack([c2u_Wr, c2r_Wr])

    agg2, cntp = _seg_with_counts(x2, src, dst)
    inv2 = _cnt_inv(cntp)
    h1 = _layer_relu(agg2, x2, inv2, wl1, bl1, wr1)
    agg2b = _seg_plain(h1, src, dst)
    h2 = _layer_lin(agg2b, h1, inv2, wl2, bl2, wr2)

    ia = jnp.concatenate([edge_label_index[0],
                          jnp.zeros((LP - L,), jnp.int32)])
    ib = jnp.concatenate([edge_label_index[1],
                          jnp.zeros((LP - L,), jnp.int32)])
    pred = _cls(h2, ia, ib)
    return pred[:L]


# speedup vs baseline: 3.6977x; 1.3916x over previous
import jax
import jax.numpy as jnp
from jax import lax
from jax.experimental import pallas as pl
from jax.experimental.pallas import tpu as pltpu
from jax.experimental.pallas import tpu_sc as plsc

H = 128
NU = 10000
NR = 10000
E = 320000
L = 100000

NC = 2
NS = 16

EK = 80
EPT = E // NS
SEK = 4000
NSUP = EPT // SEK
NPAIR = SEK // (2 * EK)
NPAD = 10240
RPT = NPAD // NS
assert EPT % EK == 0 and RPT % EK == 0

LP = 100352
LPT = LP // (NC * NS)
CK = 112
CCH = LPT // CK
assert LPT % CK == 0

_f32 = jnp.float32


def _zero_rows(ref, nrows, ncols):
    zero = jnp.zeros((16,), _f32)

    def body(r, c):
        for j in range(ncols // 16):
            ref[r, pl.ds(j * 16, 16)] = zero
        return c

    lax.fori_loop(0, nrows, body, 0)


def _fill_ones(ref, nrows):
    one = jnp.ones((16,), _f32)

    def body(r, c):
        ref[r, :] = one
        return c

    lax.fori_loop(0, nrows, body, 0)


def _make_seg(with_counts):
    def body(*refs):
        if with_counts:
            (x2, src_h, dst_h, agg2_h, cntp_h,
             rows_a, rows_b, ivb, sidx_a, sidx_b, gidx_a, gidx_b, cnt_v,
             agg_s, sem_a, sem_b) = refs
        else:
            (x2, src_h, dst_h, agg2_h,
             rows_a, rows_b, ivb, sidx_a, sidx_b, gidx_a, gidx_b,
             agg_s, sem_a, sem_b) = refs
            cntp_h = cnt_v = None

        cid = lax.axis_index("c")
        sid = lax.axis_index("s")

        _zero_rows(rows_a, EK, H)
        row0 = sid * RPT
        for j in range(RPT // EK):
            pltpu.sync_copy(rows_a, agg_s.at[pl.ds(row0 + j * EK, EK)])
        if with_counts:
            zero = jnp.zeros((16,), _f32)

            def zc(r, c):
                cnt_v[pl.ds(r * 16, 16)] = zero
                return c

            lax.fori_loop(0, NPAD // 16, zc, 0)
        plsc.subcore_barrier()

        base_e = sid * EPT
        cidv = jnp.zeros((16,), jnp.int32) + cid
        one16 = jnp.ones((16,), _f32)

        def blend(oa, sidx_ref, gidx_ref):
            def cp(k, c2):
                sv16 = ivb[0, 0, pl.ds(oa + k * 16, 16)]
                dv16 = ivb[1, 0, pl.ds(oa + k * 16, 16)]
                d16 = (dv16 - sv16) * cidv
                sidx16 = sv16 + d16
                sidx_ref[pl.ds(k * 16, 16)] = sidx16
                gidx_ref[pl.ds(k * 16, 16)] = dv16 - d16
                if with_counts:
                    plsc.addupdate_scatter(cnt_v, [sidx16], one16)
                return c2

            lax.fori_loop(0, EK // 16, cp, 0)

        def gather(gidx_ref, rows_ref, sem_ref):
            return pltpu.async_copy(x2.at[1 - cid].at[gidx_ref],
                                    rows_ref, sem_ref)

        def supchunk(sc, carry):
            soff = base_e + sc * SEK
            pltpu.sync_copy(src_h.at[pl.ds(soff, SEK)], ivb.at[0, 0])
            pltpu.sync_copy(dst_h.at[pl.ds(soff, SEK)], ivb.at[1, 0])

            def pair(q, carry2):
                oa = 2 * q * EK
                ob = oa + EK
                blend(oa, sidx_a, gidx_a)
                ga = gather(gidx_a, rows_a, sem_a)
                blend(ob, sidx_b, gidx_b)
                ga.wait()
                gb = gather(gidx_b, rows_b, sem_b)
                pltpu.sync_copy(rows_a, agg_s.at[sidx_a], add=True)
                gb.wait()
                pltpu.sync_copy(rows_b, agg_s.at[sidx_b], add=True)
                return carry2

            lax.fori_loop(0, NPAIR, pair, 0)
            return carry

        lax.fori_loop(0, NSUP, supchunk, 0)
        if with_counts:
            pltpu.sync_copy(cnt_v, cntp_h.at[cid, sid])
        plsc.subcore_barrier()

        for j in range(RPT // EK):
            r = row0 + j * EK
            pltpu.sync_copy(agg_s.at[pl.ds(r, EK)], rows_a)
            pltpu.sync_copy(rows_a, agg2_h.at[cid, pl.ds(r, EK)])

    out_type = [jax.ShapeDtypeStruct((2, NPAD, H), _f32)]
    scratch = [
        pltpu.VMEM((EK, H), _f32),
        pltpu.VMEM((EK, H), _f32),
        pltpu.VMEM((2, 1, SEK), jnp.int32),
        pltpu.VMEM((EK,), jnp.int32),
        pltpu.VMEM((EK,), jnp.int32),
        pltpu.VMEM((EK,), jnp.int32),
        pltpu.VMEM((EK,), jnp.int32),
    ]
    if with_counts:
        out_type += [jax.ShapeDtypeStruct((2, NS, NPAD), _f32)]
        scratch += [pltpu.VMEM((NPAD,), _f32)]
    scratch += [pltpu.VMEM_SHARED((NPAD, H), _f32)]
    scratch += [pltpu.SemaphoreType.DMA, pltpu.SemaphoreType.DMA]
    return pl.kernel(
        body,
        out_type=out_type if with_counts else out_type[0],
        mesh=plsc.VectorSubcoreMesh(core_axis_name="c", subcore_axis_name="s"),
        scratch_types=scratch,
        compiler_params=pltpu.CompilerParams(needs_layout_passes=False),
        name="seg_sum_counts" if with_counts else "seg_sum",
    )


_seg_with_counts = _make_seg(True)
_seg_plain = _make_seg(False)


def _cls_body(h2_h, ia_h, ib_h, pred_h, a_v, b_v, iav, ibv, out_v,
              sem_a, sem_b):
    cid = lax.axis_index("c")
    sid = lax.axis_index("s")
    wid = sid * NC + cid
    base = wid * LPT

    def chunk(c, carry):
        off = base + c * CK
        pltpu.sync_copy(ia_h.at[pl.ds(off, CK)], iav)
        pltpu.sync_copy(ib_h.at[pl.ds(off, CK)], ibv)
        ca = pltpu.async_copy(h2_h.at[0].at[iav], a_v, sem_a)
        cb = pltpu.async_copy(h2_h.at[1].at[ibv], b_v, sem_b)
        ca.wait()
        cb.wait()

        def grp(g, carry2):
            rows = g * 16 + lax.iota(jnp.int32, 16)
            acc = jnp.zeros((16,), _f32)
            for f in range(H):
                cols = jnp.full((16,), f, jnp.int32)
                av = plsc.load_gather(a_v, [rows, cols])
                bv = plsc.load_gather(b_v, [rows, cols])
                acc = acc + av * bv
            out_v[pl.ds(c * CK + g * 16, 16)] = acc
            return carry2

        lax.fori_loop(0, CK // 16, grp, 0)
        return carry

    lax.fori_loop(0, CCH, chunk, 0)
    pltpu.sync_copy(out_v, pred_h.at[pl.ds(base, LPT)])


_cls = pl.kernel(
    _cls_body,
    out_type=jax.ShapeDtypeStruct((LP,), _f32),
    mesh=plsc.VectorSubcoreMesh(core_axis_name="c", subcore_axis_name="s"),
    scratch_types=[
        pltpu.VMEM((CK, H), _f32),
        pltpu.VMEM((CK, H), _f32),
        pltpu.VMEM((CK,), jnp.int32),
        pltpu.VMEM((CK,), jnp.int32),
        pltpu.VMEM((LPT,), _f32),
        pltpu.SemaphoreType.DMA,
        pltpu.SemaphoreType.DMA,
    ],
    compiler_params=pltpu.CompilerParams(needs_layout_passes=False),
    name="edge_dot_classifier",
)


BS = 1000


def _x2_body(ue_ref, rx_ref, remb_ref, lw_ref, lb_ref, o_ref):
    o_ref[0] = ue_ref[...]
    acc = lb_ref[...] + remb_ref[...]
    rx = rx_ref[...]
    lw = lw_ref[...]
    for k in range(10):
        acc = acc + rx[:, k:k + 1] * lw[k:k + 1, :]
    o_ref[1] = acc


_x2_proj = pl.pallas_call(
    _x2_body,
    grid=(NR // BS,),
    in_specs=[
        pl.BlockSpec((BS, H), lambda i: (i, 0)),
        pl.BlockSpec((BS, 10), lambda i: (i, 0)),
        pl.BlockSpec((BS, H), lambda i: (i, 0)),
        pl.BlockSpec((10, H), lambda i: (0, 0)),
        pl.BlockSpec((1, H), lambda i: (0, 0)),
    ],
    out_specs=pl.BlockSpec((2, BS, H), lambda i: (0, i, 0)),
    out_shape=jax.ShapeDtypeStruct((2, NR, H), _f32),
)


CB = 1024


def _cnt_inv_body(c_ref, o_ref):
    s = jnp.sum(c_ref[0], axis=0)
    o_ref[0] = (1.0 / jnp.maximum(s, 1.0))[:, None]


_cnt_inv = pl.pallas_call(
    _cnt_inv_body,
    grid=(2, NPAD // CB),
    in_specs=[pl.BlockSpec((1, NS, CB), lambda j, i: (j, 0, i))],
    out_specs=pl.BlockSpec((1, CB, 1), lambda j, i: (j, i, 0)),
    out_shape=jax.ShapeDtypeStruct((2, NPAD, 1), _f32),
)


def _make_layer(relu):
    def body(a_ref, x_ref, c_ref, wl_ref, bl_ref, wr_ref, o_ref):
        inv = c_ref[0]
        h = (jnp.dot(a_ref[0] * inv, wl_ref[0], preferred_element_type=_f32)
             + bl_ref[0]
             + jnp.dot(x_ref[0], wr_ref[0], preferred_element_type=_f32))
        if relu:
            h = jnp.maximum(h, 0.0)
        o_ref[0] = h

    return pl.pallas_call(
        body,
        grid=(2, NU // BS),
        in_specs=[
            pl.BlockSpec((1, BS, H), lambda j, i: (j, i, 0)),
            pl.BlockSpec((1, BS, H), lambda j, i: (j, i, 0)),
            pl.BlockSpec((1, BS, 1), lambda j, i: (j, i, 0)),
            pl.BlockSpec((1, H, H), lambda j, i: (j, 0, 0)),
            pl.BlockSpec((1, 1, H), lambda j, i: (j, 0, 0)),
            pl.BlockSpec((1, H, H), lambda j, i: (j, 0, 0)),
        ],
        out_specs=pl.BlockSpec((1, BS, H), lambda j, i: (j, i, 0)),
        out_shape=jax.ShapeDtypeStruct((2, NU, H), _f32),
    )


_layer_relu = _make_layer(True)
_layer_lin = _make_layer(False)


def kernel(user_node_id, recipe_node_id, recipe_x, edge_index_review,
           edge_label_index, user_emb, recipe_emb, lin_w, lin_b,
           c1u_Wl, c1u_bl, c1u_Wr, c1r_Wl, c1r_bl, c1r_Wr,
           c2u_Wl, c2u_bl, c2u_Wr, c2r_Wl, c2r_bl, c2r_Wr):
    src = edge_index_review[0]
    dst = edge_index_review[1]

    x2 = _x2_proj(user_emb, recipe_x, recipe_emb, lin_w, lin_b.reshape(1, H))

    wl1 = jnp.stack([c1u_Wl, c1r_Wl])
    bl1 = jnp.stack([c1u_bl, c1r_bl]).reshape(2, 1, H)
    wr1 = jnp.stack([c1u_Wr, c1r_Wr])
    wl2 = jnp.stack([c2u_Wl, c2r_Wl])
    bl2 = jnp.stack([c2u_bl, c2r_bl]).reshape(2, 1, H)
    wr2 = jnp.stack([c2u_Wr, c2r_Wr])

    agg2, cntp = _seg_with_counts(x2, src, dst)
    inv2 = _cnt_inv(cntp)
    h1 = _layer_relu(agg2, x2, inv2, wl1, bl1, wr1)
    agg2b = _seg_plain(h1, src, dst)
    h2 = _layer_lin(agg2b, h1, inv2, wl2, bl2, wr2)

    ia = jnp.concatenate([edge_label_index[0],
                          jnp.zeros((LP - L,), jnp.int32)])
    ib = jnp.concatenate([edge_label_index[1],
                          jnp.zeros((LP - L,), jnp.int32)])
    pred = _cls(h2, ia, ib)
    return pred[:L]
